# baseline (speedup 1.0000x reference)
import functools

import jax
import jax.numpy as jnp
from jax.experimental import pallas as pl

_TILE_N = 1024


def _upsample_kernel(xq_ref, sx_ref, sf_ref, out_ref, *, S):
    d2 = (
        (xq_ref[:, 0:1] - sx_ref[0:1, :]) ** 2
        + (xq_ref[:, 1:2] - sx_ref[1:2, :]) ** 2
        + (xq_ref[:, 2:3] - sx_ref[2:3, :]) ** 2
    )

    iota = jax.lax.broadcasted_iota(jnp.int32, d2.shape, 1)
    wsum = jnp.zeros((d2.shape[0], 1), jnp.float32)
    wmat = jnp.zeros(d2.shape, jnp.float32)
    d = d2
    for k in range(3):
        mk = jnp.min(d, axis=1, keepdims=True)
        cand = jnp.where(d == mk, iota, S)
        first = jnp.min(cand, axis=1, keepdims=True)
        onehot = iota == first
        w = 1.0 / jnp.maximum(jnp.sqrt(mk), 1e-10)
        wmat = jnp.where(onehot, w, wmat)
        wsum = wsum + w
        if k < 2:
            d = jnp.where(onehot, jnp.inf, d)

    out = jax.lax.dot_general(
        wmat, sf_ref[...],
        dimension_numbers=(((1,), (0,)), ((), ())),
        precision=jax.lax.Precision.HIGHEST,
        preferred_element_type=jnp.float32,
    )
    out_ref[...] = jnp.clip(out / wsum, -100.0, 100.0)


def kernel(xyz, sparse_xyz, sparse_flow):
    B, C, N = xyz.shape
    S = sparse_xyz.shape[2]
    nt = N // _TILE_N

    xq = jnp.transpose(xyz, (0, 2, 1)).reshape(B * N, C)
    sx = jnp.transpose(sparse_xyz, (1, 0, 2)).reshape(C, B * S)
    sf = jnp.transpose(sparse_flow, (0, 2, 1)).reshape(B * S, C)

    out = pl.pallas_call(
        functools.partial(_upsample_kernel, S=S),
        grid=(B, nt),
        in_specs=[
            pl.BlockSpec((_TILE_N, C), lambda b, t: (b * nt + t, 0)),
            pl.BlockSpec((C, S), lambda b, t: (0, b)),
            pl.BlockSpec((S, C), lambda b, t: (b, 0)),
        ],
        out_specs=pl.BlockSpec((_TILE_N, C), lambda b, t: (b * nt + t, 0)),
        out_shape=jax.ShapeDtypeStruct((B * N, C), jnp.float32),
    )(xq, sx, sf)

    return jnp.transpose(out.reshape(B, N, C), (0, 2, 1))


# speedup vs baseline: 1.0563x; 1.0563x over previous
import functools

import jax
import jax.numpy as jnp
from jax.experimental import pallas as pl

_TILE_N = 1024


def _upsample_kernel(xyz_ref, sxt_ref, sft_ref, out_ref, *, S):
    d2 = (
        (sxt_ref[0, :, 0:1] - xyz_ref[0, 0:1, :]) ** 2
        + (sxt_ref[0, :, 1:2] - xyz_ref[0, 1:2, :]) ** 2
        + (sxt_ref[0, :, 2:3] - xyz_ref[0, 2:3, :]) ** 2
    )

    iota = jax.lax.broadcasted_iota(jnp.int32, d2.shape, 0)
    wsum = jnp.zeros((1, d2.shape[1]), jnp.float32)
    acc = [jnp.zeros((1, d2.shape[1]), jnp.float32) for _ in range(3)]
    d = d2
    for k in range(3):
        mk = jnp.min(d, axis=0, keepdims=True)
        cand = jnp.where(d == mk, iota, S)
        first = jnp.min(cand, axis=0, keepdims=True)
        onehot = iota == first
        w = 1.0 / jnp.maximum(jnp.sqrt(mk), 1e-10)
        for c in range(3):
            f = jnp.sum(jnp.where(onehot, sft_ref[0, :, c : c + 1], 0.0),
                        axis=0, keepdims=True)
            acc[c] = acc[c] + w * f
        wsum = wsum + w
        if k < 2:
            d = jnp.where(onehot, jnp.inf, d)

    for c in range(3):
        out_ref[0, c : c + 1, :] = jnp.clip(acc[c] / wsum, -100.0, 100.0)


def kernel(xyz, sparse_xyz, sparse_flow):
    B, C, N = xyz.shape
    S = sparse_xyz.shape[2]
    nt = N // _TILE_N

    sxt = jnp.transpose(sparse_xyz, (0, 2, 1))
    sft = jnp.transpose(sparse_flow, (0, 2, 1))

    return pl.pallas_call(
        functools.partial(_upsample_kernel, S=S),
        grid=(B, nt),
        in_specs=[
            pl.BlockSpec((1, C, _TILE_N), lambda b, t: (b, 0, t)),
            pl.BlockSpec((1, S, C), lambda b, t: (b, 0, 0)),
            pl.BlockSpec((1, S, C), lambda b, t: (b, 0, 0)),
        ],
        out_specs=pl.BlockSpec((1, C, _TILE_N), lambda b, t: (b, 0, t)),
        out_shape=jax.ShapeDtypeStruct((B, C, N), jnp.float32),
    )(xyz, sxt, sft)


# speedup vs baseline: 1.4616x; 1.3837x over previous
import functools

import jax
import jax.numpy as jnp
from jax.experimental import pallas as pl

_TILE_N = 1024


def _upsample_kernel(xyz_ref, sxt_ref, sft_ref, out_ref, *, S):
    d2 = (
        (sxt_ref[0, :, 0:1] - xyz_ref[0, 0:1, :]) ** 2
        + (sxt_ref[0, :, 1:2] - xyz_ref[0, 1:2, :]) ** 2
        + (sxt_ref[0, :, 2:3] - xyz_ref[0, 2:3, :]) ** 2
    )

    iota_f = jax.lax.broadcasted_iota(jnp.int32, d2.shape, 0).astype(
        jnp.float32)
    wsum = jnp.zeros((1, d2.shape[1]), jnp.float32)
    wmat = jnp.zeros(d2.shape, jnp.float32)
    d = d2
    for k in range(3):
        mk = jnp.min(d, axis=0, keepdims=True)
        cand = jnp.where(d == mk, iota_f, float(S))
        first = jnp.min(cand, axis=0, keepdims=True)
        hit = cand == first
        w = 1.0 / jnp.maximum(jnp.sqrt(mk), 1e-10)
        wmat = jnp.where(hit, w, wmat)
        wsum = wsum + w
        if k < 2:
            d = jnp.where(hit, jnp.inf, d)

    for c in range(3):
        f = jnp.sum(wmat * sft_ref[0, :, c : c + 1], axis=0, keepdims=True)
        out_ref[0, c : c + 1, :] = jnp.clip(f / wsum, -100.0, 100.0)


def kernel(xyz, sparse_xyz, sparse_flow):
    B, C, N = xyz.shape
    S = sparse_xyz.shape[2]
    nt = N // _TILE_N

    sxt = jnp.transpose(sparse_xyz, (0, 2, 1))
    sft = jnp.transpose(sparse_flow, (0, 2, 1))

    return pl.pallas_call(
        functools.partial(_upsample_kernel, S=S),
        grid=(B, nt),
        in_specs=[
            pl.BlockSpec((1, C, _TILE_N), lambda b, t: (b, 0, t)),
            pl.BlockSpec((1, S, C), lambda b, t: (b, 0, 0)),
            pl.BlockSpec((1, S, C), lambda b, t: (b, 0, 0)),
        ],
        out_specs=pl.BlockSpec((1, C, _TILE_N), lambda b, t: (b, 0, t)),
        out_shape=jax.ShapeDtypeStruct((B, C, N), jnp.float32),
    )(xyz, sxt, sft)


# speedup vs baseline: 1.5270x; 1.0448x over previous
import functools

import jax
import jax.numpy as jnp
from jax.experimental import pallas as pl

_TILE_N = 1024


def _upsample_kernel(xyz_ref, sx_ref, sf_ref, out_ref, *, S):
    sxc = [sx_ref[0, c : c + 1, :].reshape(S, 1) for c in range(3)]
    sfc = [sf_ref[0, c : c + 1, :].reshape(S, 1) for c in range(3)]
    d2 = (
        (sxc[0] - xyz_ref[0, 0:1, :]) ** 2
        + (sxc[1] - xyz_ref[0, 1:2, :]) ** 2
        + (sxc[2] - xyz_ref[0, 2:3, :]) ** 2
    )

    iota_f = jax.lax.broadcasted_iota(jnp.int32, d2.shape, 0).astype(
        jnp.float32)
    wsum = jnp.zeros((1, d2.shape[1]), jnp.float32)
    wmat = jnp.zeros(d2.shape, jnp.float32)
    d = d2
    for k in range(3):
        mk = jnp.min(d, axis=0, keepdims=True)
        cand = jnp.where(d == mk, iota_f, float(S))
        first = jnp.min(cand, axis=0, keepdims=True)
        hit = cand == first
        w = 1.0 / jnp.maximum(jnp.sqrt(mk), 1e-10)
        wmat = jnp.where(hit, w, wmat)
        wsum = wsum + w
        if k < 2:
            d = jnp.where(hit, jnp.inf, d)

    for c in range(3):
        f = jnp.sum(wmat * sfc[c], axis=0, keepdims=True)
        out_ref[0, c : c + 1, :] = jnp.clip(f / wsum, -100.0, 100.0)


def kernel(xyz, sparse_xyz, sparse_flow):
    B, C, N = xyz.shape
    S = sparse_xyz.shape[2]
    nt = N // _TILE_N

    return pl.pallas_call(
        functools.partial(_upsample_kernel, S=S),
        grid=(B, nt),
        in_specs=[
            pl.BlockSpec((1, C, _TILE_N), lambda b, t: (b, 0, t)),
            pl.BlockSpec((1, C, S), lambda b, t: (b, 0, 0)),
            pl.BlockSpec((1, C, S), lambda b, t: (b, 0, 0)),
        ],
        out_specs=pl.BlockSpec((1, C, _TILE_N), lambda b, t: (b, 0, t)),
        out_shape=jax.ShapeDtypeStruct((B, C, N), jnp.float32),
    )(xyz, sparse_xyz, sparse_flow)


# speedup vs baseline: 1.5575x; 1.0200x over previous
import functools

import jax
import jax.numpy as jnp
from jax.experimental import pallas as pl

_TILE_N = 1024


def _upsample_kernel(xyz_ref, sx_ref, sf_ref, out_ref, *, S):
    sxc = [sx_ref[0, c : c + 1, :].reshape(S, 1) for c in range(3)]
    sfc = [sf_ref[0, c : c + 1, :].reshape(S, 1) for c in range(3)]
    d2 = (
        (sxc[0] - xyz_ref[0, 0:1, :]) ** 2
        + (sxc[1] - xyz_ref[0, 1:2, :]) ** 2
        + (sxc[2] - xyz_ref[0, 2:3, :]) ** 2
    )

    iota = jax.lax.broadcasted_iota(jnp.int32, d2.shape, 0)
    wsum = jnp.zeros((1, d2.shape[1]), jnp.float32)
    wmat = jnp.zeros(d2.shape, jnp.float32)
    d = d2
    for k in range(3):
        mk = jnp.min(d, axis=0, keepdims=True)
        first = jnp.argmin(d, axis=0, keepdims=True)
        hit = iota == first
        w = 1.0 / jnp.maximum(jnp.sqrt(mk), 1e-10)
        wmat = jnp.where(hit, w, wmat)
        wsum = wsum + w
        if k < 2:
            d = jnp.where(hit, jnp.inf, d)

    for c in range(3):
        f = jnp.sum(wmat * sfc[c], axis=0, keepdims=True)
        out_ref[0, c : c + 1, :] = jnp.clip(f / wsum, -100.0, 100.0)


def kernel(xyz, sparse_xyz, sparse_flow):
    B, C, N = xyz.shape
    S = sparse_xyz.shape[2]
    nt = N // _TILE_N

    return pl.pallas_call(
        functools.partial(_upsample_kernel, S=S),
        grid=(B, nt),
        in_specs=[
            pl.BlockSpec((1, C, _TILE_N), lambda b, t: (b, 0, t)),
            pl.BlockSpec((1, C, S), lambda b, t: (b, 0, 0)),
            pl.BlockSpec((1, C, S), lambda b, t: (b, 0, 0)),
        ],
        out_specs=pl.BlockSpec((1, C, _TILE_N), lambda b, t: (b, 0, t)),
        out_shape=jax.ShapeDtypeStruct((B, C, N), jnp.float32),
    )(xyz, sparse_xyz, sparse_flow)
